# baseline (speedup 1.0000x reference)
import math

import jax
import jax.numpy as jnp
from jax import lax
from jax.experimental import pallas as pl
from jax.experimental.pallas import tpu as pltpu
from jax.experimental.pallas import tpu_sc as plsc

N_NODES = 50000
N_EDGES = 800000
EP = 819200
NW = 32
EPW = EP // NW
EPT = EP // 16
BLK = 128
NB1 = EPW // BLK
NB2 = EPT // BLK
ROWS_PT = N_NODES // 16

EB = 2048
NBE = EP // EB
NBLK = 2000
NBN = N_NODES // NBLK

INV32 = 1.0 / math.sqrt(32.0)
INV16 = 0.25
INVNN = 0.25
C_S = math.sin(math.pi / 8.0)
C_X = math.cos(math.pi / 8.0)
SQRT3 = math.sqrt(3.0)
INV_SQRT3 = 1.0 / math.sqrt(3.0)
STEP = 2.5 / 9.0
SQRT10 = math.sqrt(10.0)
INV_SQRT10 = 1.0 / math.sqrt(10.0)
INV_SQRT100 = 0.1
INV_SQRTN = 1.0 / math.sqrt(50000.0)

def _sc_mesh():
    return plsc.VectorSubcoreMesh(core_axis_name="c", subcore_axis_name="s",
                                  num_cores=2, num_subcores=16)
_F32 = jnp.float32


RING = 4
NB_C0 = 300
NB_C1 = 100
EPW0 = NB_C0 * BLK
EPW1 = NB_C1 * BLK


def _sc_gather1(pos8, h, srcp, dstp):
    def body(pos_hbm, h_hbm, src_hbm, dst_hbm, rec_hbm,
             idx_s_all, idx_d_all, ps_v, pd_v, hs_v, *sems):
        sem_g = sems[:RING]
        sem_w = sems[RING:]
        c = lax.axis_index("c")
        s = lax.axis_index("s")
        base = jnp.where(c == 0, s * EPW0, 16 * EPW0 + s * EPW1)
        nblk = jnp.where(c == 0, NB_C0, NB_C1)
        load_base = jnp.minimum(base, EP - EPW0)
        delta = base - load_base
        pltpu.sync_copy(src_hbm.at[pl.ds(load_base, EPW0)], idx_s_all)
        pltpu.sync_copy(dst_hbm.at[pl.ds(load_base, EPW0)], idx_d_all)

        def g_descs(slot, i):
            isl = pl.ds(i * BLK + delta, BLK)
            return (
                pltpu.make_async_copy(pos_hbm.at[idx_s_all.at[isl]],
                                      ps_v.at[slot], sem_g[slot]),
                pltpu.make_async_copy(pos_hbm.at[idx_d_all.at[isl]],
                                      pd_v.at[slot], sem_g[slot]),
                pltpu.make_async_copy(h_hbm.at[idx_s_all.at[isl]],
                                      hs_v.at[slot], sem_g[slot]),
            )

        def w_descs(slot, i):
            off = base + i * BLK
            rows = rec_hbm.at[pl.ds(off, BLK)]
            return (
                pltpu.make_async_copy(ps_v.at[slot],
                                      rows.at[:, pl.ds(0, 8)], sem_w[slot]),
                pltpu.make_async_copy(pd_v.at[slot],
                                      rows.at[:, pl.ds(8, 8)], sem_w[slot]),
                pltpu.make_async_copy(hs_v.at[slot],
                                      rows.at[:, pl.ds(16, 32)], sem_w[slot]),
            )

        def step(g, carry):
            gds = []
            for b in range(RING):
                d = g_descs(b, g * RING + b)
                for x in d:
                    x.start()
                gds.append(d)
            wds = []
            for b in range(RING):
                for x in gds[b]:
                    x.wait()
                w = w_descs(b, g * RING + b)
                for x in w:
                    x.start()
                wds.append(w)
            for w in wds:
                for x in w:
                    x.wait()
            return carry

        lax.fori_loop(0, nblk // RING, step, 0)

    f = pl.kernel(
        body,
        out_type=jax.ShapeDtypeStruct((EP, 128), _F32),
        mesh=_sc_mesh(),
        compiler_params=pltpu.CompilerParams(use_tc_tiling_on_sc=False),
        scratch_types=[pltpu.VMEM((EPW0,), jnp.int32),
                       pltpu.VMEM((EPW0,), jnp.int32),
                       pltpu.VMEM((RING, BLK, 8), _F32),
                       pltpu.VMEM((RING, BLK, 8), _F32),
                       pltpu.VMEM((RING, BLK, 32), _F32)]
                      + [pltpu.SemaphoreType.DMA] * (2 * RING),
    )
    return f(pos8, h, srcp, dstp)


def _sc_scatter(pay, dstp, zer):
    def body(pay_hbm, dst_hbm, zer_hbm, out_hbm, idx_v, pay_v, acc, *sems):
        sem_l = sems[:RING]
        sem_s = sems[RING:]
        c = lax.axis_index("c")
        s = lax.axis_index("s")
        base = s * EPT

        def l_descs(slot, i, qi):
            off = base + i * BLK
            return (
                pltpu.make_async_copy(dst_hbm.at[pl.ds(off, BLK)],
                                      idx_v.at[slot], sem_l[slot]),
                pltpu.make_async_copy(
                    pay_hbm.at[pl.ds(off, BLK)].at[:, pl.ds(qi * 24, 24)],
                    pay_v.at[slot], sem_l[slot]),
            )

        def s_desc(slot):
            return pltpu.make_async_copy(pay_v.at[slot],
                                         acc.at[idx_v.at[slot]], sem_s[slot])

        for p in range(2):
            qi = 2 * p + c
            pltpu.sync_copy(zer_hbm, acc.at[pl.ds(s * ROWS_PT, ROWS_PT)])
            plsc.subcore_barrier()

            def step(g, carry):
                lds = []
                for b in range(RING):
                    d = l_descs(b, g * RING + b, qi)
                    for x in d:
                        x.start()
                    lds.append(d)
                sds = []
                for b in range(RING):
                    for x in lds[b]:
                        x.wait()
                    sd = s_desc(b)
                    sd.start(add=True)
                    sds.append(sd)
                for sd in sds:
                    sd.wait()
                return carry

            lax.fori_loop(0, NB2 // RING, step, 0)
            plsc.subcore_barrier()
            pltpu.sync_copy(
                acc.at[pl.ds(s * ROWS_PT, ROWS_PT)],
                out_hbm.at[pl.ds(s * ROWS_PT, ROWS_PT)].at[:, pl.ds(qi * 24, 24)])

    f = pl.kernel(
        body,
        out_type=jax.ShapeDtypeStruct((N_NODES, 128), _F32),
        mesh=_sc_mesh(),
        compiler_params=pltpu.CompilerParams(use_tc_tiling_on_sc=False),
        scratch_types=[pltpu.VMEM((RING, BLK), jnp.int32),
                       pltpu.VMEM((RING, BLK, 24), _F32),
                       pltpu.VMEM_SHARED((N_NODES, 24), _F32)]
                      + [pltpu.SemaphoreType.DMA] * (2 * RING),
    )
    return f(pay, dstp, zer)


def _sc_gather2(t2, srcp):
    def body(t2_hbm, src_hbm, hh_hbm, idx_all, row_v, *sems):
        sem_g = sems[:RING]
        sem_w = sems[RING:]
        c = lax.axis_index("c")
        s = lax.axis_index("s")
        base = jnp.where(c == 0, s * EPW0, 16 * EPW0 + s * EPW1)
        nblk = jnp.where(c == 0, NB_C0, NB_C1)
        load_base = jnp.minimum(base, EP - EPW0)
        delta = base - load_base
        pltpu.sync_copy(src_hbm.at[pl.ds(load_base, EPW0)], idx_all)

        def g_desc(slot, i):
            return pltpu.make_async_copy(
                t2_hbm.at[idx_all.at[pl.ds(i * BLK + delta, BLK)]],
                row_v.at[slot], sem_g[slot])

        def w_desc(slot, i):
            return pltpu.make_async_copy(
                row_v.at[slot],
                hh_hbm.at[pl.ds(base + i * BLK, BLK)].at[:, pl.ds(0, 64)],
                sem_w[slot])

        def step(g, carry):
            gds = []
            for b in range(RING):
                d = g_desc(b, g * RING + b)
                d.start()
                gds.append(d)
            wds = []
            for b in range(RING):
                gds[b].wait()
                w = w_desc(b, g * RING + b)
                w.start()
                wds.append(w)
            for w in wds:
                w.wait()
            return carry

        lax.fori_loop(0, nblk // RING, step, 0)

    f = pl.kernel(
        body,
        out_type=jax.ShapeDtypeStruct((EP, 128), _F32),
        mesh=_sc_mesh(),
        compiler_params=pltpu.CompilerParams(use_tc_tiling_on_sc=False),
        scratch_types=[pltpu.VMEM((EPW0,), jnp.int32),
                       pltpu.VMEM((RING, BLK, 64), _F32)]
                      + [pltpu.SemaphoreType.DMA] * (2 * RING),
    )
    return f(t2, srcp)


def _tc_node_linear(x, w_lin, w_sc):
    def body(x_ref, wl_ref, ws_ref, h_ref, s_ref):
        xb = x_ref[...]
        h_ref[...] = jnp.dot(xb, wl_ref[...], preferred_element_type=_F32) * INV32
        s_ref[...] = jnp.dot(xb, ws_ref[...], preferred_element_type=_F32) * INV32

    return pl.pallas_call(
        body,
        grid=(NBN,),
        in_specs=[pl.BlockSpec((NBLK, 32), lambda i: (i, 0)),
                  pl.BlockSpec((32, 32), lambda i: (0, 0)),
                  pl.BlockSpec((32, 32), lambda i: (0, 0))],
        out_specs=(pl.BlockSpec((NBLK, 32), lambda i: (i, 0)),
                   pl.BlockSpec((NBLK, 32), lambda i: (i, 0))),
        out_shape=(jax.ShapeDtypeStruct((N_NODES, 32), _F32),
                   jax.ShapeDtypeStruct((N_NODES, 32), _F32)),
    )(x, w_lin, w_sc)


def _tc_edge_math(rec1, p):
    def body(rec_ref, fw1, fw2, a0, a1, gw1, gw2,
             pay_ref, q_ref):
        rec = rec_ref[...]
        ev = rec[:, 0:3] - rec[:, 8:11]
        r = jnp.sqrt(jnp.sum(ev * ev, axis=1, keepdims=True) + 1e-12)
        sh1 = (SQRT3 / r) * ev
        centers = lax.broadcasted_iota(jnp.int32, (1, 10), 1).astype(_F32) * STEP
        emb = jnp.exp(-(((r - centers) / STEP) ** 2)) * SQRT10
        w1h = jax.nn.silu(jnp.dot(emb, fw1[...], preferred_element_type=_F32)
                          * INV_SQRT10)
        w = jnp.dot(w1h, fw2[...], preferred_element_type=_F32) * INV_SQRT100
        hsb = rec[:, 16:48]
        P0 = jnp.dot(hsb * w[:, :32], a0[...], preferred_element_type=_F32)
        A1 = jnp.dot(hsb * w[:, 32:], a1[...], preferred_element_type=_F32)
        rows = pl.program_id(0) * EB + lax.broadcasted_iota(jnp.int32, (EB, 1), 0)
        msk = (rows < N_EDGES).astype(_F32)
        pay = jnp.concatenate(
            [P0, A1 * sh1[:, 0:1], A1 * sh1[:, 1:2], A1 * sh1[:, 2:3]],
            axis=1) * msk
        zpad = jnp.zeros((EB, 4), _F32)
        pay_ref[...] = jnp.concatenate(
            [pay[:, :20], zpad, pay[:, 20:40], zpad, pay[:, 40:60], zpad,
             pay[:, 60:], zpad, jnp.zeros((EB, 32), _F32)], axis=1)
        w2h = jax.nn.silu(jnp.dot(emb, gw1[...], preferred_element_type=_F32)
                          * INV_SQRT10)
        w2 = jnp.dot(w2h, gw2[...], preferred_element_type=_F32) * INV_SQRT100
        q_ref[...] = (jnp.concatenate(
            [w2[:, :16],
             w2[:, 16:] * (sh1[:, 0:1] * INV_SQRT3),
             w2[:, 16:] * (sh1[:, 1:2] * INV_SQRT3),
             w2[:, 16:] * (sh1[:, 2:3] * INV_SQRT3)], axis=1) * msk
        ).astype(jnp.bfloat16)

    return pl.pallas_call(
        body,
        grid=(NBE,),
        in_specs=[pl.BlockSpec((EB, 128), lambda i: (i, 0)),
                  pl.BlockSpec((10, 100), lambda i: (0, 0)),
                  pl.BlockSpec((100, 64), lambda i: (0, 0)),
                  pl.BlockSpec((32, 32), lambda i: (0, 0)),
                  pl.BlockSpec((32, 16), lambda i: (0, 0)),
                  pl.BlockSpec((10, 100), lambda i: (0, 0)),
                  pl.BlockSpec((100, 32), lambda i: (0, 0))],
        out_specs=(pl.BlockSpec((EB, 128), lambda i: (i, 0)),
                   pl.BlockSpec((EB, 64), lambda i: (i, 0))),
        out_shape=(jax.ShapeDtypeStruct((EP, 128), _F32),
                   jax.ShapeDtypeStruct((EP, 64), jnp.bfloat16)),
    )(rec1, p['fc1_W1'], p['fc1_W2'], p['lin2_0e'], p['lin2_1o'],
      p['fc2_W1'], p['fc2_W2'])


def _tc_gate(macc, s, b0, b1):
    def body(acc_ref, s_ref, b0_ref, b1_ref, t2_ref, sg_ref):
        mrec = acc_ref[...]
        accb = jnp.concatenate([mrec[:, 0:20], mrec[:, 24:44],
                                mrec[:, 48:68], mrec[:, 72:92]],
                               axis=1)
        accum0 = accb[:, :32]
        accum1 = accb[:, 32:]
        y_scal = C_S * s_ref[...] + C_X * accum0 * (INV32 * INVNN)
        g_scal = jax.nn.silu(y_scal[:, :16])
        gate = jax.nn.sigmoid(y_scal[:, 16:32])
        h0 = jnp.dot(g_scal, b0_ref[...], preferred_element_type=_F32) * INV16
        parts = [h0]
        for cc in range(3):
            gc = accum1[:, 16 * cc:16 * (cc + 1)] * (INV32 * INVNN) * gate
            parts.append(jnp.dot(gc, b1_ref[...], preferred_element_type=_F32)
                         * INV16)
        t2_ref[...] = jnp.concatenate(parts, axis=1)

        @pl.when(pl.program_id(0) == 0)
        def _zero():
            sg_ref[...] = jnp.zeros_like(sg_ref)

        sg_ref[...] += jnp.sum(g_scal, axis=0, keepdims=True)

    return pl.pallas_call(
        body,
        grid=(NBN,),
        in_specs=[pl.BlockSpec((NBLK, 128), lambda i: (i, 0)),
                  pl.BlockSpec((NBLK, 32), lambda i: (i, 0)),
                  pl.BlockSpec((16, 16), lambda i: (0, 0)),
                  pl.BlockSpec((16, 16), lambda i: (0, 0))],
        out_specs=(pl.BlockSpec((NBLK, 64), lambda i: (i, 0)),
                   pl.BlockSpec((1, 16), lambda i: (0, 0))),
        out_shape=(jax.ShapeDtypeStruct((N_NODES, 64), _F32),
                   jax.ShapeDtypeStruct((1, 16), _F32)),
    )(macc, s, b0, b1)


def _tc_reduce(hh, q, sg, scw, lw):
    def body(hh_ref, q_ref, sg_ref, scw_ref, lw_ref, out_ref, s64_ref):
        @pl.when(pl.program_id(0) == 0)
        def _zero():
            s64_ref[...] = jnp.zeros_like(s64_ref)

        s64_ref[...] += jnp.sum(
            hh_ref[...][:, :64] * q_ref[...].astype(_F32), axis=0,
            keepdims=True)

        @pl.when(pl.program_id(0) == NBE - 1)
        def _fin():
            s64 = s64_ref[...]
            mid = jnp.concatenate(
                [s64[:, :16], s64[:, 16:32] + s64[:, 32:48] + s64[:, 48:64]],
                axis=1)
            out = jnp.dot(mid, lw_ref[...], preferred_element_type=_F32) \
                * (INV32 * INVNN)
            s2 = jnp.dot(sg_ref[...], scw_ref[...], preferred_element_type=_F32) \
                * INV16
            out_ref[...] = (C_S * s2 + C_X * out) * INV_SQRTN

    return pl.pallas_call(
        body,
        grid=(NBE,),
        in_specs=[pl.BlockSpec((EB, 128), lambda i: (i, 0)),
                  pl.BlockSpec((EB, 64), lambda i: (i, 0)),
                  pl.BlockSpec((1, 16), lambda i: (0, 0)),
                  pl.BlockSpec((16, 32), lambda i: (0, 0)),
                  pl.BlockSpec((32, 32), lambda i: (0, 0))],
        out_specs=pl.BlockSpec((1, 32), lambda i: (0, 0)),
        out_shape=jax.ShapeDtypeStruct((1, 32), _F32),
        scratch_shapes=[pltpu.VMEM((1, 64), _F32)],
    )(hh, q, sg, scw, lw)


def kernel(x, pos, params, edge_index, batch):
    src = edge_index[0]
    dst = edge_index[1]
    pad = EP - N_EDGES
    srcp = jnp.concatenate([src, jnp.zeros((pad,), jnp.int32)])
    dstp = jnp.concatenate([dst, jnp.zeros((pad,), jnp.int32)])
    pos8 = jnp.pad(pos, ((0, 0), (0, 5)))
    h, s = _tc_node_linear(x, params['lin1_W'], params['sc1_W'])
    rec1 = _sc_gather1(pos8, h, srcp, dstp)
    pay, q = _tc_edge_math(rec1, params)
    zer = jnp.zeros((ROWS_PT, 24), _F32)
    macc = _sc_scatter(pay, dstp, zer)
    t2, sg = _tc_gate(macc, s, params['lin1b_0e'], params['lin1b_1o'])
    hh = _sc_gather2(t2, srcp)
    return _tc_reduce(hh, q, sg, params['sc2_W'], params['lin2b_W'])


# speedup vs baseline: 1.0133x; 1.0133x over previous
import math

import jax
import jax.numpy as jnp
from jax import lax
from jax.experimental import pallas as pl
from jax.experimental.pallas import tpu as pltpu
from jax.experimental.pallas import tpu_sc as plsc

N_NODES = 50000
N_EDGES = 800000
EP = 819200
NW = 32
EPW = EP // NW
EPT = EP // 16
BLK = 128
NB1 = EPW // BLK
NB2 = EPT // BLK
ROWS_PT = N_NODES // 16

EB = 2048
NBE = EP // EB
NBLK = 2000
NBN = N_NODES // NBLK

INV32 = 1.0 / math.sqrt(32.0)
INV16 = 0.25
INVNN = 0.25
C_S = math.sin(math.pi / 8.0)
C_X = math.cos(math.pi / 8.0)
SQRT3 = math.sqrt(3.0)
INV_SQRT3 = 1.0 / math.sqrt(3.0)
STEP = 2.5 / 9.0
SQRT10 = math.sqrt(10.0)
INV_SQRT10 = 1.0 / math.sqrt(10.0)
INV_SQRT100 = 0.1
INV_SQRTN = 1.0 / math.sqrt(50000.0)

def _sc_mesh():
    return plsc.VectorSubcoreMesh(core_axis_name="c", subcore_axis_name="s",
                                  num_cores=2, num_subcores=16)
_F32 = jnp.float32


RING = 8


def _sc_gather1(pos8, h, srcp, dstp):
    def body(pos_hbm, h_hbm, src_hbm, dst_hbm, rec_hbm,
             idx_s_all, idx_d_all, ps_v, pd_v, hs_v, *sems):
        sem_g = sems[:RING]
        sem_w = sems[RING:]
        wid = lax.axis_index("s") * 2 + lax.axis_index("c")
        base = wid * EPW
        pltpu.sync_copy(src_hbm.at[wid], idx_s_all)
        pltpu.sync_copy(dst_hbm.at[wid], idx_d_all)

        def g_descs(slot, i):
            return (
                pltpu.make_async_copy(pos_hbm.at[idx_s_all.at[i]],
                                      ps_v.at[slot], sem_g[slot]),
                pltpu.make_async_copy(pos_hbm.at[idx_d_all.at[i]],
                                      pd_v.at[slot], sem_g[slot]),
                pltpu.make_async_copy(h_hbm.at[idx_s_all.at[i]],
                                      hs_v.at[slot], sem_g[slot]),
            )

        def w_descs(slot, i):
            off = base + i * BLK
            rows = rec_hbm.at[pl.ds(off, BLK)]
            return (
                pltpu.make_async_copy(ps_v.at[slot],
                                      rows.at[:, pl.ds(0, 8)], sem_w[slot]),
                pltpu.make_async_copy(pd_v.at[slot],
                                      rows.at[:, pl.ds(8, 8)], sem_w[slot]),
                pltpu.make_async_copy(hs_v.at[slot],
                                      rows.at[:, pl.ds(16, 32)], sem_w[slot]),
            )

        def step(g, carry):
            gds = []
            for b in range(RING):
                d = g_descs(b, g * RING + b)
                for x in d:
                    x.start()
                gds.append(d)
            wds = []
            for b in range(RING):
                for x in gds[b]:
                    x.wait()
                w = w_descs(b, g * RING + b)
                for x in w:
                    x.start()
                wds.append(w)
            for w in wds:
                for x in w:
                    x.wait()
            return carry

        lax.fori_loop(0, NB1 // RING, step, 0)

    f = pl.kernel(
        body,
        out_type=jax.ShapeDtypeStruct((EP, 128), _F32),
        mesh=_sc_mesh(),
        compiler_params=pltpu.CompilerParams(use_tc_tiling_on_sc=False),
        scratch_types=[pltpu.VMEM((NB1, BLK), jnp.int32),
                       pltpu.VMEM((NB1, BLK), jnp.int32),
                       pltpu.VMEM((RING, BLK, 8), _F32),
                       pltpu.VMEM((RING, BLK, 8), _F32),
                       pltpu.VMEM((RING, BLK, 32), _F32)]
                      + [pltpu.SemaphoreType.DMA] * (2 * RING),
    )
    return f(pos8, h, srcp, dstp)


def _sc_scatter(pay, dstp, zer):
    def body(pay_hbm, dst_hbm, zer_hbm, out_hbm, idx_v, pay_v, acc, *sems):
        sem_l = sems[:RING]
        sem_s = sems[RING:]
        c = lax.axis_index("c")
        s = lax.axis_index("s")
        base = s * EPT

        def l_descs(slot, i, qi):
            off = base + i * BLK
            return (
                pltpu.make_async_copy(dst_hbm.at[pl.ds(off, BLK)],
                                      idx_v.at[slot], sem_l[slot]),
                pltpu.make_async_copy(
                    pay_hbm.at[pl.ds(off, BLK)].at[:, pl.ds(qi * 24, 24)],
                    pay_v.at[slot], sem_l[slot]),
            )

        def s_desc(slot):
            return pltpu.make_async_copy(pay_v.at[slot],
                                         acc.at[idx_v.at[slot]], sem_s[slot])

        for p in range(2):
            qi = 2 * p + c
            pltpu.sync_copy(zer_hbm, acc.at[pl.ds(s * ROWS_PT, ROWS_PT)])
            plsc.subcore_barrier()

            def step(g, carry):
                lds = []
                for b in range(RING):
                    d = l_descs(b, g * RING + b, qi)
                    for x in d:
                        x.start()
                    lds.append(d)
                sds = []
                for b in range(RING):
                    for x in lds[b]:
                        x.wait()
                    sd = s_desc(b)
                    sd.start(add=True)
                    sds.append(sd)
                for sd in sds:
                    sd.wait()
                return carry

            lax.fori_loop(0, NB2 // RING, step, 0)
            plsc.subcore_barrier()
            pltpu.sync_copy(
                acc.at[pl.ds(s * ROWS_PT, ROWS_PT)],
                out_hbm.at[pl.ds(s * ROWS_PT, ROWS_PT)].at[:, pl.ds(qi * 24, 24)])

    f = pl.kernel(
        body,
        out_type=jax.ShapeDtypeStruct((N_NODES, 128), _F32),
        mesh=_sc_mesh(),
        compiler_params=pltpu.CompilerParams(use_tc_tiling_on_sc=False),
        scratch_types=[pltpu.VMEM((RING, BLK), jnp.int32),
                       pltpu.VMEM((RING, BLK, 24), _F32),
                       pltpu.VMEM_SHARED((N_NODES, 24), _F32)]
                      + [pltpu.SemaphoreType.DMA] * (2 * RING),
    )
    return f(pay, dstp, zer)


def _sc_gather2(t2, srcp):
    def body(t2_hbm, src_hbm, hh_hbm, idx_all, row_v, *sems):
        sem_g = sems[:RING]
        sem_w = sems[RING:]
        wid = lax.axis_index("s") * 2 + lax.axis_index("c")
        base = wid * EPW
        pltpu.sync_copy(src_hbm.at[wid], idx_all)

        def g_desc(slot, i):
            return pltpu.make_async_copy(
                t2_hbm.at[idx_all.at[i]],
                row_v.at[slot], sem_g[slot])

        def w_desc(slot, i):
            return pltpu.make_async_copy(
                row_v.at[slot],
                hh_hbm.at[pl.ds(base + i * BLK, BLK)].at[:, pl.ds(0, 64)],
                sem_w[slot])

        def step(g, carry):
            gds = []
            for b in range(RING):
                d = g_desc(b, g * RING + b)
                d.start()
                gds.append(d)
            wds = []
            for b in range(RING):
                gds[b].wait()
                w = w_desc(b, g * RING + b)
                w.start()
                wds.append(w)
            for w in wds:
                w.wait()
            return carry

        lax.fori_loop(0, NB1 // RING, step, 0)

    f = pl.kernel(
        body,
        out_type=jax.ShapeDtypeStruct((EP, 128), _F32),
        mesh=_sc_mesh(),
        compiler_params=pltpu.CompilerParams(use_tc_tiling_on_sc=False),
        scratch_types=[pltpu.VMEM((NB1, BLK), jnp.int32),
                       pltpu.VMEM((RING, BLK, 64), _F32)]
                      + [pltpu.SemaphoreType.DMA] * (2 * RING),
    )
    return f(t2, srcp)


def _tc_node_linear(x, w_lin, w_sc):
    def body(x_ref, wl_ref, ws_ref, h_ref, s_ref):
        xb = x_ref[...]
        h_ref[...] = jnp.dot(xb, wl_ref[...], preferred_element_type=_F32) * INV32
        s_ref[...] = jnp.dot(xb, ws_ref[...], preferred_element_type=_F32) * INV32

    return pl.pallas_call(
        body,
        grid=(NBN,),
        in_specs=[pl.BlockSpec((NBLK, 32), lambda i: (i, 0)),
                  pl.BlockSpec((32, 32), lambda i: (0, 0)),
                  pl.BlockSpec((32, 32), lambda i: (0, 0))],
        out_specs=(pl.BlockSpec((NBLK, 32), lambda i: (i, 0)),
                   pl.BlockSpec((NBLK, 32), lambda i: (i, 0))),
        out_shape=(jax.ShapeDtypeStruct((N_NODES, 32), _F32),
                   jax.ShapeDtypeStruct((N_NODES, 32), _F32)),
    )(x, w_lin, w_sc)


def _edge_geom(rec, n10):
    ev = rec[:, 0:3] - rec[:, 8:11]
    r = jnp.sqrt(jnp.sum(ev * ev, axis=1, keepdims=True) + 1e-12)
    sh1 = (SQRT3 / r) * ev
    centers = lax.broadcasted_iota(jnp.int32, (1, n10), 1).astype(_F32) * STEP
    emb = jnp.exp(-(((r - centers) / STEP) ** 2)) * SQRT10
    return sh1, emb


def _tc_edge_pay(rec1, p):
    def body(rec_ref, fw1, fw2, a0, a1, pay_ref):
        rec = rec_ref[...]
        sh1, emb = _edge_geom(rec, 10)
        w1h = jax.nn.silu(jnp.dot(emb, fw1[...], preferred_element_type=_F32)
                          * INV_SQRT10)
        w = jnp.dot(w1h, fw2[...], preferred_element_type=_F32) * INV_SQRT100
        hsb = rec[:, 16:48]
        P0 = jnp.dot(hsb * w[:, :32], a0[...], preferred_element_type=_F32)
        A1 = jnp.dot(hsb * w[:, 32:], a1[...], preferred_element_type=_F32)
        rows = pl.program_id(0) * EB + lax.broadcasted_iota(jnp.int32, (EB, 1), 0)
        msk = (rows < N_EDGES).astype(_F32)
        pay = jnp.concatenate(
            [P0, A1 * sh1[:, 0:1], A1 * sh1[:, 1:2], A1 * sh1[:, 2:3]],
            axis=1) * msk
        zpad = jnp.zeros((EB, 4), _F32)
        pay_ref[...] = jnp.concatenate(
            [pay[:, :20], zpad, pay[:, 20:40], zpad, pay[:, 40:60], zpad,
             pay[:, 60:], zpad, jnp.zeros((EB, 32), _F32)], axis=1)

    return pl.pallas_call(
        body,
        grid=(NBE,),
        in_specs=[pl.BlockSpec((EB, 128), lambda i: (i, 0)),
                  pl.BlockSpec((10, 100), lambda i: (0, 0)),
                  pl.BlockSpec((100, 64), lambda i: (0, 0)),
                  pl.BlockSpec((32, 32), lambda i: (0, 0)),
                  pl.BlockSpec((32, 16), lambda i: (0, 0))],
        out_specs=pl.BlockSpec((EB, 128), lambda i: (i, 0)),
        out_shape=jax.ShapeDtypeStruct((EP, 128), _F32),
    )(rec1, p['fc1_W1'], p['fc1_W2'], p['lin2_0e'], p['lin2_1o'])


def _tc_edge_q(rec1, p):
    def body(rec_ref, gw1, gw2, q_ref):
        rec = rec_ref[...]
        sh1, emb = _edge_geom(rec, 10)
        rows = pl.program_id(0) * EB + lax.broadcasted_iota(jnp.int32, (EB, 1), 0)
        msk = (rows < N_EDGES).astype(_F32)
        w2h = jax.nn.silu(jnp.dot(emb, gw1[...], preferred_element_type=_F32)
                          * INV_SQRT10)
        w2 = jnp.dot(w2h, gw2[...], preferred_element_type=_F32) * INV_SQRT100
        q_ref[...] = jnp.concatenate(
            [w2[:, :16],
             w2[:, 16:] * (sh1[:, 0:1] * INV_SQRT3),
             w2[:, 16:] * (sh1[:, 1:2] * INV_SQRT3),
             w2[:, 16:] * (sh1[:, 2:3] * INV_SQRT3)], axis=1) * msk

    return pl.pallas_call(
        body,
        grid=(NBE,),
        in_specs=[pl.BlockSpec((EB, 128), lambda i: (i, 0)),
                  pl.BlockSpec((10, 100), lambda i: (0, 0)),
                  pl.BlockSpec((100, 32), lambda i: (0, 0))],
        out_specs=pl.BlockSpec((EB, 64), lambda i: (i, 0)),
        out_shape=jax.ShapeDtypeStruct((EP, 64), _F32),
    )(rec1, p['fc2_W1'], p['fc2_W2'])


def _tc_gate(macc, s, b0, b1):
    def body(acc_ref, s_ref, b0_ref, b1_ref, t2_ref, sg_ref):
        mrec = acc_ref[...]
        accb = jnp.concatenate([mrec[:, 0:20], mrec[:, 24:44],
                                mrec[:, 48:68], mrec[:, 72:92]],
                               axis=1)
        accum0 = accb[:, :32]
        accum1 = accb[:, 32:]
        y_scal = C_S * s_ref[...] + C_X * accum0 * (INV32 * INVNN)
        g_scal = jax.nn.silu(y_scal[:, :16])
        gate = jax.nn.sigmoid(y_scal[:, 16:32])
        h0 = jnp.dot(g_scal, b0_ref[...], preferred_element_type=_F32) * INV16
        parts = [h0]
        for cc in range(3):
            gc = accum1[:, 16 * cc:16 * (cc + 1)] * (INV32 * INVNN) * gate
            parts.append(jnp.dot(gc, b1_ref[...], preferred_element_type=_F32)
                         * INV16)
        t2_ref[...] = jnp.concatenate(parts, axis=1)

        @pl.when(pl.program_id(0) == 0)
        def _zero():
            sg_ref[...] = jnp.zeros_like(sg_ref)

        sg_ref[...] += jnp.sum(g_scal, axis=0, keepdims=True)

    return pl.pallas_call(
        body,
        grid=(NBN,),
        in_specs=[pl.BlockSpec((NBLK, 128), lambda i: (i, 0)),
                  pl.BlockSpec((NBLK, 32), lambda i: (i, 0)),
                  pl.BlockSpec((16, 16), lambda i: (0, 0)),
                  pl.BlockSpec((16, 16), lambda i: (0, 0))],
        out_specs=(pl.BlockSpec((NBLK, 64), lambda i: (i, 0)),
                   pl.BlockSpec((1, 16), lambda i: (0, 0))),
        out_shape=(jax.ShapeDtypeStruct((N_NODES, 64), _F32),
                   jax.ShapeDtypeStruct((1, 16), _F32)),
    )(macc, s, b0, b1)


def _tc_reduce(hh, q, sg, scw, lw):
    def body(hh_ref, q_ref, sg_ref, scw_ref, lw_ref, out_ref, s64_ref):
        @pl.when(pl.program_id(0) == 0)
        def _zero():
            s64_ref[...] = jnp.zeros_like(s64_ref)

        s64_ref[...] += jnp.sum(hh_ref[...][:, :64] * q_ref[...], axis=0,
                                keepdims=True)

        @pl.when(pl.program_id(0) == NBE - 1)
        def _fin():
            s64 = s64_ref[...]
            mid = jnp.concatenate(
                [s64[:, :16], s64[:, 16:32] + s64[:, 32:48] + s64[:, 48:64]],
                axis=1)
            out = jnp.dot(mid, lw_ref[...], preferred_element_type=_F32) \
                * (INV32 * INVNN)
            s2 = jnp.dot(sg_ref[...], scw_ref[...], preferred_element_type=_F32) \
                * INV16
            out_ref[...] = (C_S * s2 + C_X * out) * INV_SQRTN

    return pl.pallas_call(
        body,
        grid=(NBE,),
        in_specs=[pl.BlockSpec((EB, 128), lambda i: (i, 0)),
                  pl.BlockSpec((EB, 64), lambda i: (i, 0)),
                  pl.BlockSpec((1, 16), lambda i: (0, 0)),
                  pl.BlockSpec((16, 32), lambda i: (0, 0)),
                  pl.BlockSpec((32, 32), lambda i: (0, 0))],
        out_specs=pl.BlockSpec((1, 32), lambda i: (0, 0)),
        out_shape=jax.ShapeDtypeStruct((1, 32), _F32),
        scratch_shapes=[pltpu.VMEM((1, 64), _F32)],
    )(hh, q, sg, scw, lw)


def kernel(x, pos, params, edge_index, batch):
    src = edge_index[0]
    dst = edge_index[1]
    pad = EP - N_EDGES
    srcp = jnp.concatenate([src, jnp.zeros((pad,), jnp.int32)])
    dstp = jnp.concatenate([dst, jnp.zeros((pad,), jnp.int32)])
    pos8 = jnp.pad(pos, ((0, 0), (0, 5)))
    src3 = srcp.reshape(NW, NB1, BLK)
    dst3 = dstp.reshape(NW, NB1, BLK)
    h, s = _tc_node_linear(x, params['lin1_W'], params['sc1_W'])
    rec1 = _sc_gather1(pos8, h, src3, dst3)
    pay = _tc_edge_pay(rec1, params)
    zer = jnp.zeros((ROWS_PT, 24), _F32)
    macc = _sc_scatter(pay, dstp, zer)
    q = _tc_edge_q(rec1, params)
    t2, sg = _tc_gate(macc, s, params['lin1b_0e'], params['lin1b_1o'])
    hh = _sc_gather2(t2, src3)
    return _tc_reduce(hh, q, sg, params['sc2_W'], params['lin2b_W'])


# speedup vs baseline: 1.0653x; 1.0513x over previous
import math

import jax
import jax.numpy as jnp
from jax import lax
from jax.experimental import pallas as pl
from jax.experimental.pallas import tpu as pltpu
from jax.experimental.pallas import tpu_sc as plsc

N_NODES = 50000
N_EDGES = 800000
EP = 819200
NW = 32
EPW = EP // NW
EPT = EP // 16
BLK = 128
NB1 = EPW // BLK
NB2 = EPT // BLK
ROWS_PT = N_NODES // 16

EB = 4096
NBE = EP // EB
NBLK = 2000
NBN = N_NODES // NBLK

INV32 = 1.0 / math.sqrt(32.0)
INV16 = 0.25
INVNN = 0.25
C_S = math.sin(math.pi / 8.0)
C_X = math.cos(math.pi / 8.0)
SQRT3 = math.sqrt(3.0)
INV_SQRT3 = 1.0 / math.sqrt(3.0)
STEP = 2.5 / 9.0
SQRT10 = math.sqrt(10.0)
INV_SQRT10 = 1.0 / math.sqrt(10.0)
INV_SQRT100 = 0.1
INV_SQRTN = 1.0 / math.sqrt(50000.0)

def _sc_mesh():
    return plsc.VectorSubcoreMesh(core_axis_name="c", subcore_axis_name="s",
                                  num_cores=2, num_subcores=16)
_F32 = jnp.float32


RING = 8


def _sc_gather1(pos8, h, srcp, dstp):
    def body(pos_hbm, h_hbm, src_hbm, dst_hbm, rec_hbm,
             idx_s_all, idx_d_all, ps_v, pd_v, hs_v, *sems):
        sem_g = sems[:RING]
        sem_w = sems[RING:]
        wid = lax.axis_index("s") * 2 + lax.axis_index("c")
        base = wid * EPW
        pltpu.sync_copy(src_hbm.at[wid], idx_s_all)
        pltpu.sync_copy(dst_hbm.at[wid], idx_d_all)

        def g_descs(slot, i):
            return (
                pltpu.make_async_copy(pos_hbm.at[idx_s_all.at[i]],
                                      ps_v.at[slot], sem_g[slot]),
                pltpu.make_async_copy(pos_hbm.at[idx_d_all.at[i]],
                                      pd_v.at[slot], sem_g[slot]),
                pltpu.make_async_copy(h_hbm.at[idx_s_all.at[i]],
                                      hs_v.at[slot], sem_g[slot]),
            )

        def w_descs(slot, i):
            off = base + i * BLK
            rows = rec_hbm.at[pl.ds(off, BLK)]
            return (
                pltpu.make_async_copy(ps_v.at[slot],
                                      rows.at[:, pl.ds(0, 8)], sem_w[slot]),
                pltpu.make_async_copy(pd_v.at[slot],
                                      rows.at[:, pl.ds(8, 8)], sem_w[slot]),
                pltpu.make_async_copy(hs_v.at[slot],
                                      rows.at[:, pl.ds(16, 32)], sem_w[slot]),
            )

        def step(g, carry):
            gds = []
            for b in range(RING):
                d = g_descs(b, g * RING + b)
                for x in d:
                    x.start()
                gds.append(d)
            wds = []
            for b in range(RING):
                for x in gds[b]:
                    x.wait()
                w = w_descs(b, g * RING + b)
                for x in w:
                    x.start()
                wds.append(w)
            for w in wds:
                for x in w:
                    x.wait()
            return carry

        lax.fori_loop(0, NB1 // RING, step, 0)

    f = pl.kernel(
        body,
        out_type=jax.ShapeDtypeStruct((EP, 128), _F32),
        mesh=_sc_mesh(),
        compiler_params=pltpu.CompilerParams(use_tc_tiling_on_sc=False),
        scratch_types=[pltpu.VMEM((NB1, BLK), jnp.int32),
                       pltpu.VMEM((NB1, BLK), jnp.int32),
                       pltpu.VMEM((RING, BLK, 8), _F32),
                       pltpu.VMEM((RING, BLK, 8), _F32),
                       pltpu.VMEM((RING, BLK, 32), _F32)]
                      + [pltpu.SemaphoreType.DMA] * (2 * RING),
    )
    return f(pos8, h, srcp, dstp)


def _sc_scatter(pay, dstp, zer):
    def body(pay_hbm, dst_hbm, zer_hbm, out_hbm, idx_v, pay_v, acc, *sems):
        sem_l = sems[:RING]
        sem_s = sems[RING:]
        c = lax.axis_index("c")
        s = lax.axis_index("s")
        base = s * EPT

        def l_descs(slot, i, qi):
            off = base + i * BLK
            return (
                pltpu.make_async_copy(dst_hbm.at[pl.ds(off, BLK)],
                                      idx_v.at[slot], sem_l[slot]),
                pltpu.make_async_copy(
                    pay_hbm.at[pl.ds(off, BLK)].at[:, pl.ds(qi * 24, 24)],
                    pay_v.at[slot], sem_l[slot]),
            )

        def s_desc(slot):
            return pltpu.make_async_copy(pay_v.at[slot],
                                         acc.at[idx_v.at[slot]], sem_s[slot])

        for p in range(2):
            qi = 2 * p + c
            pltpu.sync_copy(zer_hbm, acc.at[pl.ds(s * ROWS_PT, ROWS_PT)])
            plsc.subcore_barrier()

            def step(g, carry):
                lds = []
                for b in range(RING):
                    d = l_descs(b, g * RING + b, qi)
                    for x in d:
                        x.start()
                    lds.append(d)
                sds = []
                for b in range(RING):
                    for x in lds[b]:
                        x.wait()
                    sd = s_desc(b)
                    sd.start(add=True)
                    sds.append(sd)
                for sd in sds:
                    sd.wait()
                return carry

            lax.fori_loop(0, NB2 // RING, step, 0)
            plsc.subcore_barrier()
            pltpu.sync_copy(
                acc.at[pl.ds(s * ROWS_PT, ROWS_PT)],
                out_hbm.at[pl.ds(s * ROWS_PT, ROWS_PT)].at[:, pl.ds(qi * 24, 24)])

    f = pl.kernel(
        body,
        out_type=jax.ShapeDtypeStruct((N_NODES, 128), _F32),
        mesh=_sc_mesh(),
        compiler_params=pltpu.CompilerParams(use_tc_tiling_on_sc=False),
        scratch_types=[pltpu.VMEM((RING, BLK), jnp.int32),
                       pltpu.VMEM((RING, BLK, 24), _F32),
                       pltpu.VMEM_SHARED((N_NODES, 24), _F32)]
                      + [pltpu.SemaphoreType.DMA] * (2 * RING),
    )
    return f(pay, dstp, zer)


def _sc_gather2(t2, srcp):
    def body(t2_hbm, src_hbm, hh_hbm, idx_all, row_v, *sems):
        sem_g = sems[:RING]
        sem_w = sems[RING:]
        wid = lax.axis_index("s") * 2 + lax.axis_index("c")
        base = wid * EPW
        pltpu.sync_copy(src_hbm.at[wid], idx_all)

        def g_desc(slot, i):
            return pltpu.make_async_copy(
                t2_hbm.at[idx_all.at[i]],
                row_v.at[slot], sem_g[slot])

        def w_desc(slot, i):
            return pltpu.make_async_copy(
                row_v.at[slot],
                hh_hbm.at[pl.ds(base + i * BLK, BLK)].at[:, pl.ds(0, 64)],
                sem_w[slot])

        def step(g, carry):
            gds = []
            for b in range(RING):
                d = g_desc(b, g * RING + b)
                d.start()
                gds.append(d)
            wds = []
            for b in range(RING):
                gds[b].wait()
                w = w_desc(b, g * RING + b)
                w.start()
                wds.append(w)
            for w in wds:
                w.wait()
            return carry

        lax.fori_loop(0, NB1 // RING, step, 0)

    f = pl.kernel(
        body,
        out_type=jax.ShapeDtypeStruct((EP, 128), _F32),
        mesh=_sc_mesh(),
        compiler_params=pltpu.CompilerParams(use_tc_tiling_on_sc=False),
        scratch_types=[pltpu.VMEM((NB1, BLK), jnp.int32),
                       pltpu.VMEM((RING, BLK, 64), _F32)]
                      + [pltpu.SemaphoreType.DMA] * (2 * RING),
    )
    return f(t2, srcp)


def _tc_node_linear(x, w_lin, w_sc):
    def body(x_ref, wl_ref, ws_ref, h_ref, s_ref):
        xb = x_ref[...]
        h_ref[...] = jnp.dot(xb, wl_ref[...], preferred_element_type=_F32) * INV32
        s_ref[...] = jnp.dot(xb, ws_ref[...], preferred_element_type=_F32) * INV32

    return pl.pallas_call(
        body,
        grid=(NBN,),
        in_specs=[pl.BlockSpec((NBLK, 32), lambda i: (i, 0)),
                  pl.BlockSpec((32, 32), lambda i: (0, 0)),
                  pl.BlockSpec((32, 32), lambda i: (0, 0))],
        out_specs=(pl.BlockSpec((NBLK, 32), lambda i: (i, 0)),
                   pl.BlockSpec((NBLK, 32), lambda i: (i, 0))),
        out_shape=(jax.ShapeDtypeStruct((N_NODES, 32), _F32),
                   jax.ShapeDtypeStruct((N_NODES, 32), _F32)),
    )(x, w_lin, w_sc)


def _edge_geom(rec, n10):
    ev = rec[:, 0:3] - rec[:, 8:11]
    r = jnp.sqrt(jnp.sum(ev * ev, axis=1, keepdims=True) + 1e-12)
    sh1 = (SQRT3 / r) * ev
    centers = lax.broadcasted_iota(jnp.int32, (1, n10), 1).astype(_F32) * STEP
    emb = jnp.exp(-(((r - centers) / STEP) ** 2)) * SQRT10
    return sh1, emb


def _tc_edge_pay(rec1, p):
    def body(rec_ref, fw1, fw2, a0, a1, pay_ref):
        rec = rec_ref[...]
        sh1, emb = _edge_geom(rec, 10)
        w1h = jax.nn.silu(jnp.dot(emb, fw1[...], preferred_element_type=_F32)
                          * INV_SQRT10)
        w = jnp.dot(w1h, fw2[...], preferred_element_type=_F32) * INV_SQRT100
        hsb = rec[:, 16:48]
        P0 = jnp.dot(hsb * w[:, :32], a0[...], preferred_element_type=_F32)
        A1 = jnp.dot(hsb * w[:, 32:], a1[...], preferred_element_type=_F32)
        rows = pl.program_id(0) * EB + lax.broadcasted_iota(jnp.int32, (EB, 1), 0)
        msk = (rows < N_EDGES).astype(_F32)
        pay = jnp.concatenate(
            [P0, A1 * sh1[:, 0:1], A1 * sh1[:, 1:2], A1 * sh1[:, 2:3]],
            axis=1) * msk
        zpad = jnp.zeros((EB, 4), _F32)
        pay_ref[...] = jnp.concatenate(
            [pay[:, :20], zpad, pay[:, 20:40], zpad, pay[:, 40:60], zpad,
             pay[:, 60:], zpad, jnp.zeros((EB, 32), _F32)], axis=1)

    return pl.pallas_call(
        body,
        grid=(NBE,),
        in_specs=[pl.BlockSpec((EB, 128), lambda i: (i, 0)),
                  pl.BlockSpec((10, 100), lambda i: (0, 0)),
                  pl.BlockSpec((100, 64), lambda i: (0, 0)),
                  pl.BlockSpec((32, 32), lambda i: (0, 0)),
                  pl.BlockSpec((32, 16), lambda i: (0, 0))],
        out_specs=pl.BlockSpec((EB, 128), lambda i: (i, 0)),
        out_shape=jax.ShapeDtypeStruct((EP, 128), _F32),
    )(rec1, p['fc1_W1'], p['fc1_W2'], p['lin2_0e'], p['lin2_1o'])


def _tc_edge_q(rec1, p):
    def body(rec_ref, gw1, gw2, q_ref):
        rec = rec_ref[...]
        sh1, emb = _edge_geom(rec, 10)
        rows = pl.program_id(0) * EB + lax.broadcasted_iota(jnp.int32, (EB, 1), 0)
        msk = (rows < N_EDGES).astype(_F32)
        w2h = jax.nn.silu(jnp.dot(emb, gw1[...], preferred_element_type=_F32)
                          * INV_SQRT10)
        w2 = jnp.dot(w2h, gw2[...], preferred_element_type=_F32) * INV_SQRT100
        q_ref[...] = jnp.concatenate(
            [w2[:, :16],
             w2[:, 16:] * (sh1[:, 0:1] * INV_SQRT3),
             w2[:, 16:] * (sh1[:, 1:2] * INV_SQRT3),
             w2[:, 16:] * (sh1[:, 2:3] * INV_SQRT3)], axis=1) * msk

    return pl.pallas_call(
        body,
        grid=(NBE,),
        in_specs=[pl.BlockSpec((EB, 128), lambda i: (i, 0)),
                  pl.BlockSpec((10, 100), lambda i: (0, 0)),
                  pl.BlockSpec((100, 32), lambda i: (0, 0))],
        out_specs=pl.BlockSpec((EB, 64), lambda i: (i, 0)),
        out_shape=jax.ShapeDtypeStruct((EP, 64), _F32),
    )(rec1, p['fc2_W1'], p['fc2_W2'])


def _tc_gate(macc, s, b0, b1):
    def body(acc_ref, s_ref, b0_ref, b1_ref, t2_ref, sg_ref):
        mrec = acc_ref[...]
        accb = jnp.concatenate([mrec[:, 0:20], mrec[:, 24:44],
                                mrec[:, 48:68], mrec[:, 72:92]],
                               axis=1)
        accum0 = accb[:, :32]
        accum1 = accb[:, 32:]
        y_scal = C_S * s_ref[...] + C_X * accum0 * (INV32 * INVNN)
        g_scal = jax.nn.silu(y_scal[:, :16])
        gate = jax.nn.sigmoid(y_scal[:, 16:32])
        h0 = jnp.dot(g_scal, b0_ref[...], preferred_element_type=_F32) * INV16
        parts = [h0]
        for cc in range(3):
            gc = accum1[:, 16 * cc:16 * (cc + 1)] * (INV32 * INVNN) * gate
            parts.append(jnp.dot(gc, b1_ref[...], preferred_element_type=_F32)
                         * INV16)
        t2_ref[...] = jnp.concatenate(parts, axis=1)

        @pl.when(pl.program_id(0) == 0)
        def _zero():
            sg_ref[...] = jnp.zeros_like(sg_ref)

        sg_ref[...] += jnp.sum(g_scal, axis=0, keepdims=True)

    return pl.pallas_call(
        body,
        grid=(NBN,),
        in_specs=[pl.BlockSpec((NBLK, 128), lambda i: (i, 0)),
                  pl.BlockSpec((NBLK, 32), lambda i: (i, 0)),
                  pl.BlockSpec((16, 16), lambda i: (0, 0)),
                  pl.BlockSpec((16, 16), lambda i: (0, 0))],
        out_specs=(pl.BlockSpec((NBLK, 64), lambda i: (i, 0)),
                   pl.BlockSpec((1, 16), lambda i: (0, 0))),
        out_shape=(jax.ShapeDtypeStruct((N_NODES, 64), _F32),
                   jax.ShapeDtypeStruct((1, 16), _F32)),
    )(macc, s, b0, b1)


def _tc_reduce(hh, q, sg, scw, lw):
    def body(hh_ref, q_ref, sg_ref, scw_ref, lw_ref, out_ref, s64_ref):
        @pl.when(pl.program_id(0) == 0)
        def _zero():
            s64_ref[...] = jnp.zeros_like(s64_ref)

        s64_ref[...] += jnp.sum(hh_ref[...][:, :64] * q_ref[...], axis=0,
                                keepdims=True)

        @pl.when(pl.program_id(0) == NBE - 1)
        def _fin():
            s64 = s64_ref[...]
            mid = jnp.concatenate(
                [s64[:, :16], s64[:, 16:32] + s64[:, 32:48] + s64[:, 48:64]],
                axis=1)
            out = jnp.dot(mid, lw_ref[...], preferred_element_type=_F32) \
                * (INV32 * INVNN)
            s2 = jnp.dot(sg_ref[...], scw_ref[...], preferred_element_type=_F32) \
                * INV16
            out_ref[...] = (C_S * s2 + C_X * out) * INV_SQRTN

    return pl.pallas_call(
        body,
        grid=(NBE,),
        in_specs=[pl.BlockSpec((EB, 128), lambda i: (i, 0)),
                  pl.BlockSpec((EB, 64), lambda i: (i, 0)),
                  pl.BlockSpec((1, 16), lambda i: (0, 0)),
                  pl.BlockSpec((16, 32), lambda i: (0, 0)),
                  pl.BlockSpec((32, 32), lambda i: (0, 0))],
        out_specs=pl.BlockSpec((1, 32), lambda i: (0, 0)),
        out_shape=jax.ShapeDtypeStruct((1, 32), _F32),
        scratch_shapes=[pltpu.VMEM((1, 64), _F32)],
    )(hh, q, sg, scw, lw)


def kernel(x, pos, params, edge_index, batch):
    src = edge_index[0]
    dst = edge_index[1]
    pad = EP - N_EDGES
    srcp = jnp.concatenate([src, jnp.zeros((pad,), jnp.int32)])
    dstp = jnp.concatenate([dst, jnp.zeros((pad,), jnp.int32)])
    pos8 = jnp.pad(pos, ((0, 0), (0, 5)))
    src3 = srcp.reshape(NW, NB1, BLK)
    dst3 = dstp.reshape(NW, NB1, BLK)
    h, s = _tc_node_linear(x, params['lin1_W'], params['sc1_W'])
    rec1 = _sc_gather1(pos8, h, src3, dst3)
    pay = _tc_edge_pay(rec1, params)
    zer = jnp.zeros((ROWS_PT, 24), _F32)
    macc = _sc_scatter(pay, dstp, zer)
    q = _tc_edge_q(rec1, params)
    t2, sg = _tc_gate(macc, s, params['lin1b_0e'], params['lin1b_1o'])
    hh = _sc_gather2(t2, src3)
    return _tc_reduce(hh, q, sg, params['sc2_W'], params['lin2b_W'])


# speedup vs baseline: 1.0800x; 1.0137x over previous
import math

import jax
import jax.numpy as jnp
from jax import lax
from jax.experimental import pallas as pl
from jax.experimental.pallas import tpu as pltpu
from jax.experimental.pallas import tpu_sc as plsc

N_NODES = 50000
N_EDGES = 800000
EP = 819200
NW = 32
EPW = EP // NW
EPT = EP // 16
BLK = 128
NB1 = EPW // BLK
NB2 = EPT // BLK
ROWS_PT = N_NODES // 16

EB = 8192
NBE = EP // EB
NBLK = 2000
NBN = N_NODES // NBLK

INV32 = 1.0 / math.sqrt(32.0)
INV16 = 0.25
INVNN = 0.25
C_S = math.sin(math.pi / 8.0)
C_X = math.cos(math.pi / 8.0)
SQRT3 = math.sqrt(3.0)
INV_SQRT3 = 1.0 / math.sqrt(3.0)
STEP = 2.5 / 9.0
SQRT10 = math.sqrt(10.0)
INV_SQRT10 = 1.0 / math.sqrt(10.0)
INV_SQRT100 = 0.1
INV_SQRTN = 1.0 / math.sqrt(50000.0)

def _sc_mesh():
    return plsc.VectorSubcoreMesh(core_axis_name="c", subcore_axis_name="s",
                                  num_cores=2, num_subcores=16)
_F32 = jnp.float32


RING = 8


def _sc_gather1(pos8, h, srcp, dstp):
    def body(pos_hbm, h_hbm, src_hbm, dst_hbm, rec_hbm,
             idx_s_all, idx_d_all, ps_v, pd_v, hs_v, *sems):
        sem_g = sems[:RING]
        sem_w = sems[RING:]
        wid = lax.axis_index("s") * 2 + lax.axis_index("c")
        base = wid * EPW
        pltpu.sync_copy(src_hbm.at[wid], idx_s_all)
        pltpu.sync_copy(dst_hbm.at[wid], idx_d_all)

        def g_descs(slot, i):
            return (
                pltpu.make_async_copy(pos_hbm.at[idx_s_all.at[i]],
                                      ps_v.at[slot], sem_g[slot]),
                pltpu.make_async_copy(pos_hbm.at[idx_d_all.at[i]],
                                      pd_v.at[slot], sem_g[slot]),
                pltpu.make_async_copy(h_hbm.at[idx_s_all.at[i]],
                                      hs_v.at[slot], sem_g[slot]),
            )

        def w_descs(slot, i):
            off = base + i * BLK
            rows = rec_hbm.at[pl.ds(off, BLK)]
            return (
                pltpu.make_async_copy(ps_v.at[slot],
                                      rows.at[:, pl.ds(0, 8)], sem_w[slot]),
                pltpu.make_async_copy(pd_v.at[slot],
                                      rows.at[:, pl.ds(8, 8)], sem_w[slot]),
                pltpu.make_async_copy(hs_v.at[slot],
                                      rows.at[:, pl.ds(16, 32)], sem_w[slot]),
            )

        def step(g, carry):
            gds = []
            for b in range(RING):
                d = g_descs(b, g * RING + b)
                for x in d:
                    x.start()
                gds.append(d)
            wds = []
            for b in range(RING):
                for x in gds[b]:
                    x.wait()
                w = w_descs(b, g * RING + b)
                for x in w:
                    x.start()
                wds.append(w)
            for w in wds:
                for x in w:
                    x.wait()
            return carry

        lax.fori_loop(0, NB1 // RING, step, 0)

    f = pl.kernel(
        body,
        out_type=jax.ShapeDtypeStruct((EP, 128), _F32),
        mesh=_sc_mesh(),
        compiler_params=pltpu.CompilerParams(use_tc_tiling_on_sc=False),
        scratch_types=[pltpu.VMEM((NB1, BLK), jnp.int32),
                       pltpu.VMEM((NB1, BLK), jnp.int32),
                       pltpu.VMEM((RING, BLK, 8), _F32),
                       pltpu.VMEM((RING, BLK, 8), _F32),
                       pltpu.VMEM((RING, BLK, 32), _F32)]
                      + [pltpu.SemaphoreType.DMA] * (2 * RING),
    )
    return f(pos8, h, srcp, dstp)


def _sc_scatter(pay, dstp, zer):
    def body(pay_hbm, dst_hbm, zer_hbm, out_hbm, idx_v, pay_v, acc, *sems):
        sem_l = sems[:RING]
        sem_s = sems[RING:]
        c = lax.axis_index("c")
        s = lax.axis_index("s")
        base = s * EPT

        def l_descs(slot, i, qi):
            off = base + i * BLK
            return (
                pltpu.make_async_copy(dst_hbm.at[pl.ds(off, BLK)],
                                      idx_v.at[slot], sem_l[slot]),
                pltpu.make_async_copy(
                    pay_hbm.at[pl.ds(off, BLK)].at[:, pl.ds(qi * 24, 24)],
                    pay_v.at[slot], sem_l[slot]),
            )

        def s_desc(slot):
            return pltpu.make_async_copy(pay_v.at[slot],
                                         acc.at[idx_v.at[slot]], sem_s[slot])

        for p in range(2):
            qi = 2 * p + c
            pltpu.sync_copy(zer_hbm, acc.at[pl.ds(s * ROWS_PT, ROWS_PT)])
            plsc.subcore_barrier()

            def step(g, carry):
                lds = []
                for b in range(RING):
                    d = l_descs(b, g * RING + b, qi)
                    for x in d:
                        x.start()
                    lds.append(d)
                sds = []
                for b in range(RING):
                    for x in lds[b]:
                        x.wait()
                    sd = s_desc(b)
                    sd.start(add=True)
                    sds.append(sd)
                for sd in sds:
                    sd.wait()
                return carry

            lax.fori_loop(0, NB2 // RING, step, 0)
            plsc.subcore_barrier()
            pltpu.sync_copy(
                acc.at[pl.ds(s * ROWS_PT, ROWS_PT)],
                out_hbm.at[pl.ds(s * ROWS_PT, ROWS_PT)].at[:, pl.ds(qi * 24, 24)])

    f = pl.kernel(
        body,
        out_type=jax.ShapeDtypeStruct((N_NODES, 128), _F32),
        mesh=_sc_mesh(),
        compiler_params=pltpu.CompilerParams(use_tc_tiling_on_sc=False),
        scratch_types=[pltpu.VMEM((RING, BLK), jnp.int32),
                       pltpu.VMEM((RING, BLK, 24), _F32),
                       pltpu.VMEM_SHARED((N_NODES, 24), _F32)]
                      + [pltpu.SemaphoreType.DMA] * (2 * RING),
    )
    return f(pay, dstp, zer)


def _sc_gather2(t2, srcp):
    def body(t2_hbm, src_hbm, hh_hbm, idx_all, row_v, *sems):
        sem_g = sems[:RING]
        sem_w = sems[RING:]
        wid = lax.axis_index("s") * 2 + lax.axis_index("c")
        base = wid * EPW
        pltpu.sync_copy(src_hbm.at[wid], idx_all)

        def g_desc(slot, i):
            return pltpu.make_async_copy(
                t2_hbm.at[idx_all.at[i]],
                row_v.at[slot], sem_g[slot])

        def w_desc(slot, i):
            return pltpu.make_async_copy(
                row_v.at[slot],
                hh_hbm.at[pl.ds(base + i * BLK, BLK)].at[:, pl.ds(0, 64)],
                sem_w[slot])

        def step(g, carry):
            gds = []
            for b in range(RING):
                d = g_desc(b, g * RING + b)
                d.start()
                gds.append(d)
            wds = []
            for b in range(RING):
                gds[b].wait()
                w = w_desc(b, g * RING + b)
                w.start()
                wds.append(w)
            for w in wds:
                w.wait()
            return carry

        lax.fori_loop(0, NB1 // RING, step, 0)

    f = pl.kernel(
        body,
        out_type=jax.ShapeDtypeStruct((EP, 128), _F32),
        mesh=_sc_mesh(),
        compiler_params=pltpu.CompilerParams(use_tc_tiling_on_sc=False),
        scratch_types=[pltpu.VMEM((NB1, BLK), jnp.int32),
                       pltpu.VMEM((RING, BLK, 64), _F32)]
                      + [pltpu.SemaphoreType.DMA] * (2 * RING),
    )
    return f(t2, srcp)


def _tc_node_linear(x, w_lin, w_sc):
    def body(x_ref, wl_ref, ws_ref, h_ref, s_ref):
        xb = x_ref[...]
        h_ref[...] = jnp.dot(xb, wl_ref[...], preferred_element_type=_F32) * INV32
        s_ref[...] = jnp.dot(xb, ws_ref[...], preferred_element_type=_F32) * INV32

    return pl.pallas_call(
        body,
        grid=(NBN,),
        in_specs=[pl.BlockSpec((NBLK, 32), lambda i: (i, 0)),
                  pl.BlockSpec((32, 32), lambda i: (0, 0)),
                  pl.BlockSpec((32, 32), lambda i: (0, 0))],
        out_specs=(pl.BlockSpec((NBLK, 32), lambda i: (i, 0)),
                   pl.BlockSpec((NBLK, 32), lambda i: (i, 0))),
        out_shape=(jax.ShapeDtypeStruct((N_NODES, 32), _F32),
                   jax.ShapeDtypeStruct((N_NODES, 32), _F32)),
    )(x, w_lin, w_sc)


def _edge_geom(rec, n10):
    ev = rec[:, 0:3] - rec[:, 8:11]
    r = jnp.sqrt(jnp.sum(ev * ev, axis=1, keepdims=True) + 1e-12)
    sh1 = (SQRT3 / r) * ev
    centers = lax.broadcasted_iota(jnp.int32, (1, n10), 1).astype(_F32) * STEP
    emb = jnp.exp(-(((r - centers) / STEP) ** 2)) * SQRT10
    return sh1, emb


def _tc_edge_pay(rec1, p):
    def body(rec_ref, fw1, fw2, a0, a1, pay_ref):
        rec = rec_ref[...]
        sh1, emb = _edge_geom(rec, 10)
        w1h = jax.nn.silu(jnp.dot(emb, fw1[...], preferred_element_type=_F32)
                          * INV_SQRT10)
        w = jnp.dot(w1h, fw2[...], preferred_element_type=_F32) * INV_SQRT100
        hsb = rec[:, 16:48]
        P0 = jnp.dot(hsb * w[:, :32], a0[...], preferred_element_type=_F32)
        A1 = jnp.dot(hsb * w[:, 32:], a1[...], preferred_element_type=_F32)
        rows = pl.program_id(0) * EB + lax.broadcasted_iota(jnp.int32, (EB, 1), 0)
        msk = (rows < N_EDGES).astype(_F32)
        pay = jnp.concatenate(
            [P0, A1 * sh1[:, 0:1], A1 * sh1[:, 1:2], A1 * sh1[:, 2:3]],
            axis=1) * msk
        zpad = jnp.zeros((EB, 4), _F32)
        pay_ref[...] = jnp.concatenate(
            [pay[:, :20], zpad, pay[:, 20:40], zpad, pay[:, 40:60], zpad,
             pay[:, 60:], zpad, jnp.zeros((EB, 32), _F32)], axis=1)

    return pl.pallas_call(
        body,
        grid=(NBE,),
        in_specs=[pl.BlockSpec((EB, 128), lambda i: (i, 0)),
                  pl.BlockSpec((10, 100), lambda i: (0, 0)),
                  pl.BlockSpec((100, 64), lambda i: (0, 0)),
                  pl.BlockSpec((32, 32), lambda i: (0, 0)),
                  pl.BlockSpec((32, 16), lambda i: (0, 0))],
        out_specs=pl.BlockSpec((EB, 128), lambda i: (i, 0)),
        out_shape=jax.ShapeDtypeStruct((EP, 128), _F32),
    )(rec1, p['fc1_W1'], p['fc1_W2'], p['lin2_0e'], p['lin2_1o'])


def _tc_edge_q(rec1, p):
    def body(rec_ref, gw1, gw2, q_ref):
        rec = rec_ref[...]
        sh1, emb = _edge_geom(rec, 10)
        rows = pl.program_id(0) * EB + lax.broadcasted_iota(jnp.int32, (EB, 1), 0)
        msk = (rows < N_EDGES).astype(_F32)
        w2h = jax.nn.silu(jnp.dot(emb, gw1[...], preferred_element_type=_F32)
                          * INV_SQRT10)
        w2 = jnp.dot(w2h, gw2[...], preferred_element_type=_F32) * INV_SQRT100
        q_ref[...] = jnp.concatenate(
            [w2[:, :16],
             w2[:, 16:] * (sh1[:, 0:1] * INV_SQRT3),
             w2[:, 16:] * (sh1[:, 1:2] * INV_SQRT3),
             w2[:, 16:] * (sh1[:, 2:3] * INV_SQRT3)], axis=1) * msk

    return pl.pallas_call(
        body,
        grid=(NBE,),
        in_specs=[pl.BlockSpec((EB, 128), lambda i: (i, 0)),
                  pl.BlockSpec((10, 100), lambda i: (0, 0)),
                  pl.BlockSpec((100, 32), lambda i: (0, 0))],
        out_specs=pl.BlockSpec((EB, 64), lambda i: (i, 0)),
        out_shape=jax.ShapeDtypeStruct((EP, 64), _F32),
    )(rec1, p['fc2_W1'], p['fc2_W2'])


def _tc_gate(macc, s, b0, b1):
    def body(acc_ref, s_ref, b0_ref, b1_ref, t2_ref, sg_ref):
        mrec = acc_ref[...]
        accb = jnp.concatenate([mrec[:, 0:20], mrec[:, 24:44],
                                mrec[:, 48:68], mrec[:, 72:92]],
                               axis=1)
        accum0 = accb[:, :32]
        accum1 = accb[:, 32:]
        y_scal = C_S * s_ref[...] + C_X * accum0 * (INV32 * INVNN)
        g_scal = jax.nn.silu(y_scal[:, :16])
        gate = jax.nn.sigmoid(y_scal[:, 16:32])
        h0 = jnp.dot(g_scal, b0_ref[...], preferred_element_type=_F32) * INV16
        parts = [h0]
        for cc in range(3):
            gc = accum1[:, 16 * cc:16 * (cc + 1)] * (INV32 * INVNN) * gate
            parts.append(jnp.dot(gc, b1_ref[...], preferred_element_type=_F32)
                         * INV16)
        t2_ref[...] = jnp.concatenate(parts, axis=1)

        @pl.when(pl.program_id(0) == 0)
        def _zero():
            sg_ref[...] = jnp.zeros_like(sg_ref)

        sg_ref[...] += jnp.sum(g_scal, axis=0, keepdims=True)

    return pl.pallas_call(
        body,
        grid=(NBN,),
        in_specs=[pl.BlockSpec((NBLK, 128), lambda i: (i, 0)),
                  pl.BlockSpec((NBLK, 32), lambda i: (i, 0)),
                  pl.BlockSpec((16, 16), lambda i: (0, 0)),
                  pl.BlockSpec((16, 16), lambda i: (0, 0))],
        out_specs=(pl.BlockSpec((NBLK, 64), lambda i: (i, 0)),
                   pl.BlockSpec((1, 16), lambda i: (0, 0))),
        out_shape=(jax.ShapeDtypeStruct((N_NODES, 64), _F32),
                   jax.ShapeDtypeStruct((1, 16), _F32)),
    )(macc, s, b0, b1)


def _tc_reduce(hh, q, sg, scw, lw):
    def body(hh_ref, q_ref, sg_ref, scw_ref, lw_ref, out_ref, s64_ref):
        @pl.when(pl.program_id(0) == 0)
        def _zero():
            s64_ref[...] = jnp.zeros_like(s64_ref)

        s64_ref[...] += jnp.sum(hh_ref[...][:, :64] * q_ref[...], axis=0,
                                keepdims=True)

        @pl.when(pl.program_id(0) == NBE - 1)
        def _fin():
            s64 = s64_ref[...]
            mid = jnp.concatenate(
                [s64[:, :16], s64[:, 16:32] + s64[:, 32:48] + s64[:, 48:64]],
                axis=1)
            out = jnp.dot(mid, lw_ref[...], preferred_element_type=_F32) \
                * (INV32 * INVNN)
            s2 = jnp.dot(sg_ref[...], scw_ref[...], preferred_element_type=_F32) \
                * INV16
            out_ref[...] = (C_S * s2 + C_X * out) * INV_SQRTN

    return pl.pallas_call(
        body,
        grid=(NBE,),
        in_specs=[pl.BlockSpec((EB, 128), lambda i: (i, 0)),
                  pl.BlockSpec((EB, 64), lambda i: (i, 0)),
                  pl.BlockSpec((1, 16), lambda i: (0, 0)),
                  pl.BlockSpec((16, 32), lambda i: (0, 0)),
                  pl.BlockSpec((32, 32), lambda i: (0, 0))],
        out_specs=pl.BlockSpec((1, 32), lambda i: (0, 0)),
        out_shape=jax.ShapeDtypeStruct((1, 32), _F32),
        scratch_shapes=[pltpu.VMEM((1, 64), _F32)],
    )(hh, q, sg, scw, lw)


def kernel(x, pos, params, edge_index, batch):
    src = edge_index[0]
    dst = edge_index[1]
    pad = EP - N_EDGES
    srcp = jnp.concatenate([src, jnp.zeros((pad,), jnp.int32)])
    dstp = jnp.concatenate([dst, jnp.zeros((pad,), jnp.int32)])
    pos8 = jnp.pad(pos, ((0, 0), (0, 5)))
    src3 = srcp.reshape(NW, NB1, BLK)
    dst3 = dstp.reshape(NW, NB1, BLK)
    h, s = _tc_node_linear(x, params['lin1_W'], params['sc1_W'])
    rec1 = _sc_gather1(pos8, h, src3, dst3)
    pay = _tc_edge_pay(rec1, params)
    zer = jnp.zeros((ROWS_PT, 24), _F32)
    macc = _sc_scatter(pay, dstp, zer)
    q = _tc_edge_q(rec1, params)
    t2, sg = _tc_gate(macc, s, params['lin1b_0e'], params['lin1b_1o'])
    hh = _sc_gather2(t2, src3)
    return _tc_reduce(hh, q, sg, params['sc2_W'], params['lin2b_W'])
